# Initial kernel scaffold; baseline (speedup 1.0000x reference)
import jax
import jax.numpy as jnp
from jax.experimental import pallas as pl


def kernel(x, edge_index, mask, weight, params):
    raise NotImplementedError("write your pallas kernel here")


# speedup vs baseline: 1.2065x; 1.2065x over previous
import functools

import jax
import jax.numpy as jnp
from jax.experimental import pallas as pl
from jax.experimental.pallas import tpu as pltpu

_N = 38332
_PL = 38333
_GF = 16
_PD = 64
_C = 128
_LAYERS = 4


def _ceil_to(v, m):
    return (v + m - 1) // m * m


def _pad2(a, mp, kp):
    m, k = a.shape
    if m == mp and k == kp:
        return a
    return jnp.pad(a, ((0, mp - m), (0, kp - k)))


def _mm_body(x_ref, w_ref, o_ref):
    @pl.when(pl.program_id(2) == 0)
    def _():
        o_ref[...] = jnp.zeros_like(o_ref)

    o_ref[...] += jnp.dot(x_ref[...], w_ref[...],
                          preferred_element_type=jnp.float32)


def _mm(x, w):
    m, k = x.shape
    k2, n = w.shape
    assert k == k2
    bm = 256 if m >= 256 else _ceil_to(m, 8)
    mp = _ceil_to(m, bm)
    kp = _ceil_to(k, 128)
    bk = kp if kp <= 640 else 512
    kp = _ceil_to(kp, bk)
    np_ = _ceil_to(n, 128)
    bn = 128
    xp = _pad2(x, mp, kp)
    wp = _pad2(w, kp, np_)
    grid = (mp // bm, np_ // bn, kp // bk)
    out = pl.pallas_call(
        _mm_body,
        grid=grid,
        in_specs=[
            pl.BlockSpec((bm, bk), lambda i, j, kk: (i, kk)),
            pl.BlockSpec((bk, bn), lambda i, j, kk: (kk, j)),
        ],
        out_specs=pl.BlockSpec((bm, bn), lambda i, j, kk: (i, j)),
        out_shape=jax.ShapeDtypeStruct((mp, np_), jnp.float32),
        compiler_params=pltpu.CompilerParams(
            dimension_semantics=("arbitrary", "arbitrary", "arbitrary")),
    )(xp, wp)
    if mp != m or np_ != n:
        out = out[:m, :n]
    return out


def _colstats_body(t_ref, o_ref):
    @pl.when(pl.program_id(0) == 0)
    def _():
        o_ref[...] = jnp.zeros_like(o_ref)

    t = t_ref[...]
    s0 = jnp.sum(t, axis=0, keepdims=True)
    s1 = jnp.sum(t * t, axis=0, keepdims=True)
    o_ref[...] += jnp.concatenate(
        [s0, s1, jnp.zeros((6, t.shape[1]), jnp.float32)], axis=0)


def _colstats(t):
    m, c = t.shape
    bm = 256
    mp = _ceil_to(m, bm)
    tp = _pad2(t, mp, c)
    out = pl.pallas_call(
        _colstats_body,
        grid=(mp // bm,),
        in_specs=[pl.BlockSpec((bm, c), lambda i: (i, 0))],
        out_specs=pl.BlockSpec((8, c), lambda i: (0, 0)),
        out_shape=jax.ShapeDtypeStruct((8, c), jnp.float32),
        compiler_params=pltpu.CompilerParams(
            dimension_semantics=("arbitrary",)),
    )(tp)
    return out[0], out[1]


def _resid_body(f_ref, t_ref, s_ref, h_ref, o_ref):
    z = t_ref[...] * s_ref[0:1, :] + h_ref[0:1, :]
    o_ref[...] = f_ref[...] + jnp.where(z > 0, z, 0.01 * z)


def _resid_norm_act(feat, t, scale, shift):
    m, c = feat.shape
    bm = 256
    mp = _ceil_to(m, bm)
    fp = _pad2(feat, mp, c)
    tp = _pad2(t, mp, c)
    sc = jnp.broadcast_to(scale[None, :], (8, c))
    sh = jnp.broadcast_to(shift[None, :], (8, c))
    out = pl.pallas_call(
        _resid_body,
        grid=(mp // bm,),
        in_specs=[
            pl.BlockSpec((bm, c), lambda i: (i, 0)),
            pl.BlockSpec((bm, c), lambda i: (i, 0)),
            pl.BlockSpec((8, c), lambda i: (0, 0)),
            pl.BlockSpec((8, c), lambda i: (0, 0)),
        ],
        out_specs=pl.BlockSpec((bm, c), lambda i: (i, 0)),
        out_shape=jax.ShapeDtypeStruct((mp, c), jnp.float32),
        compiler_params=pltpu.CompilerParams(
            dimension_semantics=("arbitrary",)),
    )(fp, tp, sc, sh)
    return out[:m]


def _graph_norm_apply(feat, t, w, b, a):
    s, ss = _colstats(t)
    mean = s / t.shape[0]
    msq = ss / t.shape[0]
    var = msq - (2.0 * a - a * a) * mean * mean
    scale = jax.lax.rsqrt(var + 1e-5) * w
    shift = b - a * mean * scale
    return _resid_norm_act(feat, t, scale, shift)


def _seg_sum(vals, seg, n):
    return jax.ops.segment_sum(vals, seg, num_segments=n)


def kernel(x, edge_index, mask, weight, params):
    src, dst = edge_index[0], edge_index[1]
    half = _GF // 2
    poi = jnp.where(mask[:, :half], x[:, :half], 0.0).astype(jnp.int32)
    distance = jnp.where(mask[:, half:], 0.0, x[:, half:])
    emb_poi = jnp.take(params['emb'], poi, axis=0)
    feature = jnp.concatenate(
        [emb_poi.reshape(_N, half * _PD), distance], axis=1)

    loop = jnp.arange(_N, dtype=src.dtype)
    s_idx = jnp.concatenate([src, loop])
    d_idx = jnp.concatenate([dst, loop])
    w2 = jnp.concatenate([weight, jnp.ones((_N,), jnp.float32)])
    deg = _seg_sum(w2, d_idx, _N)
    dis = jax.lax.rsqrt(jnp.maximum(deg, 1e-12))
    norm = dis[s_idx] * w2 * dis[d_idx]

    def gcn(feat, wkey, bkey):
        xw = _mm(feat, params[wkey])
        out = _seg_sum(norm[:, None] * xw[s_idx], d_idx, _N)
        return out + params[bkey]

    def gat(feat, i):
        xw = _mm(feat, params['u%d_gat_W' % i])
        als = (xw @ params['u%d_gat_as' % i])
        ald = (xw @ params['u%d_gat_ad' % i])
        e = als[s_idx] + ald[d_idx]
        e = jnp.where(e > 0, e, 0.2 * e)
        ex = jnp.exp(e)
        ssum = _seg_sum(ex, d_idx, _N)
        alpha = ex / (ssum[d_idx] + 1e-16)
        out = _seg_sum(alpha[:, None] * xw[s_idx], d_idx, _N)
        return out + params['u%d_gat_b' % i]

    f0 = gcn(feature, 'cov_in_W', 'cov_in_b')
    feature = jnp.where(f0 > 0, f0, 0.01 * f0)
    for i in range(_LAYERS):
        t = gcn(feature, 'u%d_gcn_W' % i, 'u%d_gcn_b' % i)
        feature = _graph_norm_apply(feature, t, params['u%d_gn_w' % i],
                                    params['u%d_gn_b' % i],
                                    params['u%d_gn_a' % i])
        t = gat(feature, i)
        feature = _graph_norm_apply(feature, t, params['u%d_gn_w' % i],
                                    params['u%d_gn_b' % i],
                                    params['u%d_gn_a' % i])
    fo = gcn(feature, 'cov_out_W', 'cov_out_b')
    fo = jnp.where(fo > 0, fo, 0.01 * fo)
    v = fo.reshape(1, -1)
    h = _mm(v, params['fc1_W'])[0] + params['fc1_b']
    h = jnp.maximum(h, 0.0)
    o = _mm(h[None, :], params['fc2_W'])[0] + params['fc2_b']
    return jnp.maximum(o, 0.0)


# speedup vs baseline: 1.3449x; 1.1146x over previous
import functools

import jax
import jax.numpy as jnp
from jax import lax
from jax.experimental import pallas as pl
from jax.experimental.pallas import tpu as pltpu
from jax.experimental.pallas import tpu_sc as plsc

_N = 38332
_PL = 38333
_GF = 16
_PD = 64
_C = 128
_LAYERS = 4

_NP = 38400
_RPT = _NP // 16
_EB = 128
_EPAD = 360448
_BPT = _EPAD // (16 * _EB)


def _ceil_to(v, m):
    return (v + m - 1) // m * m


def _pad2(a, mp, kp):
    m, k = a.shape
    if m == mp and k == kp:
        return a
    return jnp.pad(a, ((0, mp - m), (0, kp - k)))


def _mm_body(x_ref, w_ref, o_ref):
    @pl.when(pl.program_id(2) == 0)
    def _():
        o_ref[...] = jnp.zeros_like(o_ref)

    o_ref[...] += jnp.dot(x_ref[...], w_ref[...],
                          preferred_element_type=jnp.float32)


def _mm(x, w):
    m, k = x.shape
    k2, n = w.shape
    assert k == k2
    bm = 256 if m >= 256 else _ceil_to(m, 8)
    mp = _ceil_to(m, bm)
    kp = _ceil_to(k, 128)
    bk = kp if kp <= 640 else 512
    kp = _ceil_to(kp, bk)
    np_ = _ceil_to(n, 128)
    bn = 128
    xp = _pad2(x, mp, kp)
    wp = _pad2(w, kp, np_)
    grid = (mp // bm, np_ // bn, kp // bk)
    out = pl.pallas_call(
        _mm_body,
        grid=grid,
        in_specs=[
            pl.BlockSpec((bm, bk), lambda i, j, kk: (i, kk)),
            pl.BlockSpec((bk, bn), lambda i, j, kk: (kk, j)),
        ],
        out_specs=pl.BlockSpec((bm, bn), lambda i, j, kk: (i, j)),
        out_shape=jax.ShapeDtypeStruct((mp, np_), jnp.float32),
        compiler_params=pltpu.CompilerParams(
            dimension_semantics=("arbitrary", "arbitrary", "arbitrary")),
    )(xp, wp)
    if mp != m or np_ != n:
        out = out[:m, :n]
    return out


def _colstats_body(t_ref, o_ref):
    @pl.when(pl.program_id(0) == 0)
    def _():
        o_ref[...] = jnp.zeros_like(o_ref)

    t = t_ref[...]
    s0 = jnp.sum(t, axis=0, keepdims=True)
    s1 = jnp.sum(t * t, axis=0, keepdims=True)
    o_ref[...] += jnp.concatenate(
        [s0, s1, jnp.zeros((6, t.shape[1]), jnp.float32)], axis=0)


def _colstats(t):
    m, c = t.shape
    bm = 256
    mp = _ceil_to(m, bm)
    tp = _pad2(t, mp, c)
    out = pl.pallas_call(
        _colstats_body,
        grid=(mp // bm,),
        in_specs=[pl.BlockSpec((bm, c), lambda i: (i, 0))],
        out_specs=pl.BlockSpec((8, c), lambda i: (0, 0)),
        out_shape=jax.ShapeDtypeStruct((8, c), jnp.float32),
        compiler_params=pltpu.CompilerParams(
            dimension_semantics=("arbitrary",)),
    )(tp)
    return out[0], out[1]


def _resid_body(f_ref, t_ref, s_ref, h_ref, o_ref):
    z = t_ref[...] * s_ref[0:1, :] + h_ref[0:1, :]
    o_ref[...] = f_ref[...] + jnp.where(z > 0, z, 0.01 * z)


def _resid_norm_act(feat, t, scale, shift):
    m, c = feat.shape
    bm = 256
    mp = _ceil_to(m, bm)
    fp = _pad2(feat, mp, c)
    tp = _pad2(t, mp, c)
    sc = jnp.broadcast_to(scale[None, :], (8, c))
    sh = jnp.broadcast_to(shift[None, :], (8, c))
    out = pl.pallas_call(
        _resid_body,
        grid=(mp // bm,),
        in_specs=[
            pl.BlockSpec((bm, c), lambda i: (i, 0)),
            pl.BlockSpec((bm, c), lambda i: (i, 0)),
            pl.BlockSpec((8, c), lambda i: (0, 0)),
            pl.BlockSpec((8, c), lambda i: (0, 0)),
        ],
        out_specs=pl.BlockSpec((bm, c), lambda i: (i, 0)),
        out_shape=jax.ShapeDtypeStruct((mp, c), jnp.float32),
        compiler_params=pltpu.CompilerParams(
            dimension_semantics=("arbitrary",)),
    )(fp, tp, sc, sh)
    return out[:m]


def _graph_norm_apply(feat, t, w, b, a):
    s, ss = _colstats(t)
    mean = s / t.shape[0]
    msq = ss / t.shape[0]
    var = msq - (2.0 * a - a * a) * mean * mean
    scale = jax.lax.rsqrt(var + 1e-5) * w
    shift = b - a * mean * scale
    return _resid_norm_act(feat, t, scale, shift)


def _seg_sum(vals, seg, n):
    return jax.ops.segment_sum(vals, seg, num_segments=n)


def _spmm_body(xw_ref, s2_ref, d2_ref, c2_ref, out_ref,
               s2v, d2v, c2v, sadj, rows, zbuf, acc, sem):
    cid = lax.axis_index("c")
    sid = lax.axis_index("s")
    tb = sid * _BPT
    pltpu.sync_copy(s2_ref.at[pl.ds(tb, _BPT)], s2v)
    pltpu.sync_copy(d2_ref.at[pl.ds(tb, _BPT)], d2v)
    pltpu.sync_copy(c2_ref.at[pl.ds(tb, _BPT)], c2v)
    z16 = jnp.zeros((16,), jnp.float32)
    for r in range(80):
        zbuf[r, pl.ds(0, 16)] = z16
    for cc in range(4):
        off = (cid * 4 + cc) * _N

        def zero_step(z, _):
            pltpu.sync_copy(zbuf, acc.at[pl.ds(sid * _RPT + z * 80, 80)])
            return _

        lax.fori_loop(0, _RPT // 80, zero_step, 0)
        plsc.subcore_barrier()

        def blk(b, _):
            for g in range(8):
                sadj[pl.ds(g * 16, 16)] = s2v[b, pl.ds(g * 16, 16)] + off
            pltpu.async_copy(xw_ref.at[sadj], rows, sem).wait()
            for g in range(8):
                c16 = c2v[b, pl.ds(g * 16, 16)]
                for e in range(16):
                    i = g * 16 + e
                    cb = lax.broadcast_in_dim(
                        lax.slice(c16, (e,), (e + 1,)), (16,), (0,))
                    rows[i, pl.ds(0, 16)] = rows[i, pl.ds(0, 16)] * cb
            pltpu.sync_copy(rows, acc.at[d2v.at[b]], add=True)
            return _

        lax.fori_loop(0, _BPT, blk, 0)
        plsc.subcore_barrier()
        pltpu.sync_copy(
            acc.at[pl.ds(sid * _RPT, _RPT)],
            out_ref.at[pl.ds((cid * 4 + cc) * _NP + sid * _RPT, _RPT)])
        plsc.subcore_barrier()


_spmm_call = functools.partial(
    pl.kernel,
    out_type=jax.ShapeDtypeStruct((8 * _NP, 16), jnp.float32),
    mesh=plsc.VectorSubcoreMesh(core_axis_name="c", subcore_axis_name="s"),
    compiler_params=pltpu.CompilerParams(use_tc_tiling_on_sc=False),
    scratch_types=[
        pltpu.VMEM((_BPT, _EB), jnp.int32),
        pltpu.VMEM((_BPT, _EB), jnp.int32),
        pltpu.VMEM((_BPT, _EB), jnp.float32),
        pltpu.VMEM((_EB,), jnp.int32),
        pltpu.VMEM((_EB, 16), jnp.float32),
        pltpu.VMEM((80, 16), jnp.float32),
        pltpu.VMEM_SHARED((_NP, 16), jnp.float32),
        pltpu.SemaphoreType.DMA,
    ],
)(_spmm_body)


def _spmm_sc(xw, s2, d2, c2):
    if xw.shape[1] != _C:
        xw = jnp.pad(xw, ((0, 0), (0, _C - xw.shape[1])))
    xw8 = xw.reshape(_N, 8, 16).transpose(1, 0, 2).reshape(8 * _N, 16)
    o = _spmm_call(xw8, s2, d2, c2)
    return o.reshape(8, _NP, 16)[:, :_N, :].transpose(1, 0, 2).reshape(_N, _C)


def kernel(x, edge_index, mask, weight, params):
    src, dst = edge_index[0], edge_index[1]
    half = _GF // 2
    poi = jnp.where(mask[:, :half], x[:, :half], 0.0).astype(jnp.int32)
    distance = jnp.where(mask[:, half:], 0.0, x[:, half:])
    emb_poi = jnp.take(params['emb'], poi, axis=0)
    feature = jnp.concatenate(
        [emb_poi.reshape(_N, half * _PD), distance], axis=1)

    loop = jnp.arange(_N, dtype=src.dtype)
    s_idx = jnp.concatenate([src, loop])
    d_idx = jnp.concatenate([dst, loop])
    w2 = jnp.concatenate([weight, jnp.ones((_N,), jnp.float32)])
    deg = _seg_sum(w2, d_idx, _N)
    dis = jax.lax.rsqrt(jnp.maximum(deg, 1e-12))
    norm = dis[s_idx] * w2 * dis[d_idx]

    npad = _EPAD - s_idx.shape[0]
    zpad_i = jnp.zeros((npad,), s_idx.dtype)
    s2 = jnp.concatenate([s_idx, zpad_i]).reshape(-1, _EB)
    d2 = jnp.concatenate([d_idx, zpad_i]).reshape(-1, _EB)

    def pad_coef(c):
        return jnp.concatenate(
            [c, jnp.zeros((npad,), jnp.float32)]).reshape(-1, _EB)

    norm2 = pad_coef(norm)

    def gcn(feat, wkey, bkey):
        xw = _mm(feat, params[wkey])
        cout = xw.shape[1]
        return _spmm_sc(xw, s2, d2, norm2)[:, :cout] + params[bkey]

    def gat(feat, i):
        xw = _mm(feat, params['u%d_gat_W' % i])
        als = (xw @ params['u%d_gat_as' % i])
        ald = (xw @ params['u%d_gat_ad' % i])
        e = als[s_idx] + ald[d_idx]
        e = jnp.where(e > 0, e, 0.2 * e)
        ex = jnp.exp(e)
        ssum = _seg_sum(ex, d_idx, _N)
        alpha = ex / (ssum[d_idx] + 1e-16)
        out = _spmm_sc(xw, s2, d2, pad_coef(alpha))
        return out + params['u%d_gat_b' % i]

    f0 = gcn(feature, 'cov_in_W', 'cov_in_b')
    feature = jnp.where(f0 > 0, f0, 0.01 * f0)
    for i in range(_LAYERS):
        t = gcn(feature, 'u%d_gcn_W' % i, 'u%d_gcn_b' % i)
        feature = _graph_norm_apply(feature, t, params['u%d_gn_w' % i],
                                    params['u%d_gn_b' % i],
                                    params['u%d_gn_a' % i])
        t = gat(feature, i)
        feature = _graph_norm_apply(feature, t, params['u%d_gn_w' % i],
                                    params['u%d_gn_b' % i],
                                    params['u%d_gn_a' % i])
    fo = gcn(feature, 'cov_out_W', 'cov_out_b')
    fo = jnp.where(fo > 0, fo, 0.01 * fo)
    v = fo.reshape(1, -1)
    h = _mm(v, params['fc1_W'])[0] + params['fc1_b']
    h = jnp.maximum(h, 0.0)
    o = _mm(h[None, :], params['fc2_W'])[0] + params['fc2_b']
    return jnp.maximum(o, 0.0)


# speedup vs baseline: 1.4177x; 1.0542x over previous
import functools

import jax
import jax.numpy as jnp
from jax import lax
from jax.experimental import pallas as pl
from jax.experimental.pallas import tpu as pltpu
from jax.experimental.pallas import tpu_sc as plsc

_N = 38332
_PL = 38333
_GF = 16
_PD = 64
_C = 128
_LAYERS = 4

_NP = 38400
_RPT = _NP // 16
_EB = 128
_EPAD = 360448
_BPT = _EPAD // (16 * _EB)


def _ceil_to(v, m):
    return (v + m - 1) // m * m


def _pad2(a, mp, kp):
    m, k = a.shape
    if m == mp and k == kp:
        return a
    return jnp.pad(a, ((0, mp - m), (0, kp - k)))


def _mm_body(x_ref, w_ref, o_ref):
    @pl.when(pl.program_id(2) == 0)
    def _():
        o_ref[...] = jnp.zeros_like(o_ref)

    o_ref[...] += jnp.dot(x_ref[...], w_ref[...],
                          preferred_element_type=jnp.float32)


def _mm(x, w):
    m, k = x.shape
    k2, n = w.shape
    assert k == k2
    bm = 256 if m >= 256 else _ceil_to(m, 8)
    mp = _ceil_to(m, bm)
    kp = _ceil_to(k, 128)
    bk = kp if kp <= 640 else 512
    kp = _ceil_to(kp, bk)
    np_ = _ceil_to(n, 128)
    bn = 128
    xp = _pad2(x, mp, kp)
    wp = _pad2(w, kp, np_)
    grid = (mp // bm, np_ // bn, kp // bk)
    out = pl.pallas_call(
        _mm_body,
        grid=grid,
        in_specs=[
            pl.BlockSpec((bm, bk), lambda i, j, kk: (i, kk)),
            pl.BlockSpec((bk, bn), lambda i, j, kk: (kk, j)),
        ],
        out_specs=pl.BlockSpec((bm, bn), lambda i, j, kk: (i, j)),
        out_shape=jax.ShapeDtypeStruct((mp, np_), jnp.float32),
        compiler_params=pltpu.CompilerParams(
            dimension_semantics=("arbitrary", "arbitrary", "arbitrary")),
    )(xp, wp)
    if mp != m or np_ != n:
        out = out[:m, :n]
    return out


def _colstats_body(t_ref, o_ref):
    @pl.when(pl.program_id(0) == 0)
    def _():
        o_ref[...] = jnp.zeros_like(o_ref)

    t = t_ref[...]
    s0 = jnp.sum(t, axis=0, keepdims=True)
    s1 = jnp.sum(t * t, axis=0, keepdims=True)
    o_ref[...] += jnp.concatenate(
        [s0, s1, jnp.zeros((6, t.shape[1]), jnp.float32)], axis=0)


def _colstats(t):
    m, c = t.shape
    bm = 256
    mp = _ceil_to(m, bm)
    tp = _pad2(t, mp, c)
    out = pl.pallas_call(
        _colstats_body,
        grid=(mp // bm,),
        in_specs=[pl.BlockSpec((bm, c), lambda i: (i, 0))],
        out_specs=pl.BlockSpec((8, c), lambda i: (0, 0)),
        out_shape=jax.ShapeDtypeStruct((8, c), jnp.float32),
        compiler_params=pltpu.CompilerParams(
            dimension_semantics=("arbitrary",)),
    )(tp)
    return out[0], out[1]


def _resid_body(f_ref, t_ref, s_ref, h_ref, o_ref):
    z = t_ref[...] * s_ref[0:1, :] + h_ref[0:1, :]
    o_ref[...] = f_ref[...] + jnp.where(z > 0, z, 0.01 * z)


def _resid_norm_act(feat, t, scale, shift):
    m, c = feat.shape
    bm = 256
    mp = _ceil_to(m, bm)
    fp = _pad2(feat, mp, c)
    tp = _pad2(t, mp, c)
    sc = jnp.broadcast_to(scale[None, :], (8, c))
    sh = jnp.broadcast_to(shift[None, :], (8, c))
    out = pl.pallas_call(
        _resid_body,
        grid=(mp // bm,),
        in_specs=[
            pl.BlockSpec((bm, c), lambda i: (i, 0)),
            pl.BlockSpec((bm, c), lambda i: (i, 0)),
            pl.BlockSpec((8, c), lambda i: (0, 0)),
            pl.BlockSpec((8, c), lambda i: (0, 0)),
        ],
        out_specs=pl.BlockSpec((bm, c), lambda i: (i, 0)),
        out_shape=jax.ShapeDtypeStruct((mp, c), jnp.float32),
        compiler_params=pltpu.CompilerParams(
            dimension_semantics=("arbitrary",)),
    )(fp, tp, sc, sh)
    return out[:m]


def _graph_norm_apply(feat, t, w, b, a):
    s, ss = _colstats(t)
    mean = s / t.shape[0]
    msq = ss / t.shape[0]
    var = msq - (2.0 * a - a * a) * mean * mean
    scale = jax.lax.rsqrt(var + 1e-5) * w
    shift = b - a * mean * scale
    return _resid_norm_act(feat, t, scale, shift)


def _seg_sum(vals, seg, n):
    return jax.ops.segment_sum(vals, seg, num_segments=n)


def _fill_sadj(sadj, s2v, b, off):
    for g in range(8):
        sadj[pl.ds(g * 16, 16)] = s2v[b, pl.ds(g * 16, 16)] + off


def _scale_rows(rows, c2v, b):
    for g in range(8):
        c16 = c2v[b, pl.ds(g * 16, 16)]
        for e in range(16):
            i = g * 16 + e
            cb = lax.broadcast_in_dim(
                lax.slice(c16, (e,), (e + 1,)), (16,), (0,))
            rows[i, pl.ds(0, 16)] = rows[i, pl.ds(0, 16)] * cb


def _spmm_body(xw_ref, s2_ref, d2_ref, c2_ref, out_ref,
               s2v, d2v, c2v, sadj_a, sadj_b, rows_a, rows_b,
               zbuf, acc, sem_a, sem_b):
    cid = lax.axis_index("c")
    sid = lax.axis_index("s")
    tb = sid * _BPT
    pltpu.sync_copy(s2_ref.at[pl.ds(tb, _BPT)], s2v)
    pltpu.sync_copy(d2_ref.at[pl.ds(tb, _BPT)], d2v)
    pltpu.sync_copy(c2_ref.at[pl.ds(tb, _BPT)], c2v)
    z16 = jnp.zeros((16,), jnp.float32)
    for r in range(80):
        zbuf[r, pl.ds(0, 16)] = z16
    for cc in range(4):
        off = (cid * 4 + cc) * _N

        def zero_step(z, _):
            pltpu.sync_copy(zbuf, acc.at[pl.ds(sid * _RPT + z * 80, 80)])
            return _

        lax.fori_loop(0, _RPT // 80, zero_step, 0)
        plsc.subcore_barrier()

        _fill_sadj(sadj_a, s2v, 0, off)
        pltpu.async_copy(xw_ref.at[sadj_a], rows_a, sem_a)
        _fill_sadj(sadj_b, s2v, 1, off)
        pltpu.async_copy(xw_ref.at[sadj_b], rows_b, sem_b)

        def blk(p, _):
            ba = 2 * p
            bb = 2 * p + 1
            pltpu.make_async_copy(xw_ref.at[sadj_a], rows_a, sem_a).wait()
            _scale_rows(rows_a, c2v, ba)
            pltpu.sync_copy(rows_a, acc.at[d2v.at[ba]], add=True)
            _fill_sadj(sadj_a, s2v, ba + 2, off)
            pltpu.async_copy(xw_ref.at[sadj_a], rows_a, sem_a)
            pltpu.make_async_copy(xw_ref.at[sadj_b], rows_b, sem_b).wait()
            _scale_rows(rows_b, c2v, bb)
            pltpu.sync_copy(rows_b, acc.at[d2v.at[bb]], add=True)
            _fill_sadj(sadj_b, s2v, bb + 2, off)
            pltpu.async_copy(xw_ref.at[sadj_b], rows_b, sem_b)
            return _

        lax.fori_loop(0, _BPT // 2 - 1, blk, 0)
        pltpu.make_async_copy(xw_ref.at[sadj_a], rows_a, sem_a).wait()
        _scale_rows(rows_a, c2v, _BPT - 2)
        pltpu.sync_copy(rows_a, acc.at[d2v.at[_BPT - 2]], add=True)
        pltpu.make_async_copy(xw_ref.at[sadj_b], rows_b, sem_b).wait()
        _scale_rows(rows_b, c2v, _BPT - 1)
        pltpu.sync_copy(rows_b, acc.at[d2v.at[_BPT - 1]], add=True)
        plsc.subcore_barrier()
        pltpu.sync_copy(
            acc.at[pl.ds(sid * _RPT, _RPT)],
            out_ref.at[pl.ds((cid * 4 + cc) * _NP + sid * _RPT, _RPT)])
        plsc.subcore_barrier()


_spmm_call = functools.partial(
    pl.kernel,
    out_type=jax.ShapeDtypeStruct((8 * _NP, 16), jnp.float32),
    mesh=plsc.VectorSubcoreMesh(core_axis_name="c", subcore_axis_name="s"),
    compiler_params=pltpu.CompilerParams(use_tc_tiling_on_sc=False),
    scratch_types=[
        pltpu.VMEM((_BPT, _EB), jnp.int32),
        pltpu.VMEM((_BPT, _EB), jnp.int32),
        pltpu.VMEM((_BPT, _EB), jnp.float32),
        pltpu.VMEM((_EB,), jnp.int32),
        pltpu.VMEM((_EB,), jnp.int32),
        pltpu.VMEM((_EB, 16), jnp.float32),
        pltpu.VMEM((_EB, 16), jnp.float32),
        pltpu.VMEM((80, 16), jnp.float32),
        pltpu.VMEM_SHARED((_NP, 16), jnp.float32),
        pltpu.SemaphoreType.DMA,
        pltpu.SemaphoreType.DMA,
    ],
)(_spmm_body)


def _spmm_sc(xw, s2, d2, c2):
    if xw.shape[1] != _C:
        xw = jnp.pad(xw, ((0, 0), (0, _C - xw.shape[1])))
    xw8 = xw.reshape(_N, 8, 16).transpose(1, 0, 2).reshape(8 * _N, 16)
    o = _spmm_call(xw8, s2, d2, c2)
    return o.reshape(8, _NP, 16)[:, :_N, :].transpose(1, 0, 2).reshape(_N, _C)


def kernel(x, edge_index, mask, weight, params):
    src, dst = edge_index[0], edge_index[1]
    half = _GF // 2
    poi = jnp.where(mask[:, :half], x[:, :half], 0.0).astype(jnp.int32)
    distance = jnp.where(mask[:, half:], 0.0, x[:, half:])
    emb_poi = jnp.take(params['emb'], poi, axis=0)
    feature = jnp.concatenate(
        [emb_poi.reshape(_N, half * _PD), distance], axis=1)

    loop = jnp.arange(_N, dtype=src.dtype)
    s_idx = jnp.concatenate([src, loop])
    d_idx = jnp.concatenate([dst, loop])
    w2 = jnp.concatenate([weight, jnp.ones((_N,), jnp.float32)])
    deg = _seg_sum(w2, d_idx, _N)
    dis = jax.lax.rsqrt(jnp.maximum(deg, 1e-12))
    norm = dis[s_idx] * w2 * dis[d_idx]

    npad = _EPAD - s_idx.shape[0]
    zpad_i = jnp.zeros((npad,), s_idx.dtype)
    s2 = jnp.concatenate([s_idx, zpad_i]).reshape(-1, _EB)
    d2 = jnp.concatenate([d_idx, zpad_i]).reshape(-1, _EB)

    def pad_coef(c):
        return jnp.concatenate(
            [c, jnp.zeros((npad,), jnp.float32)]).reshape(-1, _EB)

    norm2 = pad_coef(norm)

    def gcn(feat, wkey, bkey):
        xw = _mm(feat, params[wkey])
        cout = xw.shape[1]
        return _spmm_sc(xw, s2, d2, norm2)[:, :cout] + params[bkey]

    def gat(feat, i):
        xw = _mm(feat, params['u%d_gat_W' % i])
        als = (xw @ params['u%d_gat_as' % i])
        ald = (xw @ params['u%d_gat_ad' % i])
        e = als[s_idx] + ald[d_idx]
        e = jnp.where(e > 0, e, 0.2 * e)
        ex = jnp.exp(e)
        ssum = _seg_sum(ex, d_idx, _N)
        alpha = ex / (ssum[d_idx] + 1e-16)
        out = _spmm_sc(xw, s2, d2, pad_coef(alpha))
        return out + params['u%d_gat_b' % i]

    f0 = gcn(feature, 'cov_in_W', 'cov_in_b')
    feature = jnp.where(f0 > 0, f0, 0.01 * f0)
    for i in range(_LAYERS):
        t = gcn(feature, 'u%d_gcn_W' % i, 'u%d_gcn_b' % i)
        feature = _graph_norm_apply(feature, t, params['u%d_gn_w' % i],
                                    params['u%d_gn_b' % i],
                                    params['u%d_gn_a' % i])
        t = gat(feature, i)
        feature = _graph_norm_apply(feature, t, params['u%d_gn_w' % i],
                                    params['u%d_gn_b' % i],
                                    params['u%d_gn_a' % i])
    fo = gcn(feature, 'cov_out_W', 'cov_out_b')
    fo = jnp.where(fo > 0, fo, 0.01 * fo)
    v = fo.reshape(1, -1)
    h = _mm(v, params['fc1_W'])[0] + params['fc1_b']
    h = jnp.maximum(h, 0.0)
    o = _mm(h[None, :], params['fc2_W'])[0] + params['fc2_b']
    return jnp.maximum(o, 0.0)
